# baseline (speedup 1.0000x reference)
import functools

import jax
import jax.numpy as jnp
from jax import lax
from jax.experimental import pallas as pl
from jax.experimental.pallas import tpu as pltpu
from jax.experimental.pallas import tpu_sc as plsc

N = 10000
E = 320000
NCORE = 2
NSUB = 16
CH = 80
RB = 80
NBUF = 4


def _segsum(tables, src, dst, ea, init=None):
    col_split = len(tables) == 2
    Dh = tables[0].shape[1]
    nsl = Dh // 16
    use_init = init is not None
    ec = E if col_split else E // 2
    nch = ec // CH // NSUB
    assert nch * CH * NSUB == ec
    mesh = plsc.VectorSubcoreMesh(core_axis_name="c", subcore_axis_name="s")
    scratch = [
        [pltpu.VMEM((CH, Dh), jnp.float32) for _ in range(NBUF)],
        [pltpu.VMEM((CH,), jnp.int32) for _ in range(NBUF)],
        [pltpu.VMEM((CH,), jnp.int32) for _ in range(NBUF)],
        [pltpu.VMEM((CH,), jnp.int32) for _ in range(NBUF)],
        [pltpu.VMEM((CH,), jnp.float32) for _ in range(NBUF)],
        pltpu.VMEM_SHARED((N, Dh), jnp.float32),
        [pltpu.SemaphoreType.DMA for _ in range(NBUF)],
        [pltpu.SemaphoreType.DMA for _ in range(NBUF)],
        [pltpu.SemaphoreType.DMA for _ in range(NBUF)],
    ]

    def body(*refs):
        t_hs = refs[:len(tables)]
        refs = refs[len(tables):]
        if use_init:
            (src_h, dst_h, ea_h, init_h, out_h,
             rows, srcb, dstb, dsts, eab, aggr, sem_i, sem_g, sem_s) = refs
        else:
            (src_h, dst_h, ea_h, out_h,
             rows, srcb, dstb, dsts, eab, aggr, sem_i, sem_g, sem_s) = refs
        cid = lax.axis_index("c")
        sid = lax.axis_index("s")
        nrc = N // RB

        def _row_blocks(fn):
            for m in range((nrc + NSUB - 1) // NSUB):
                kidx = sid + NSUB * m
                if (m + 1) * NSUB <= nrc:
                    fn(kidx * RB)
                else:
                    @pl.when(kidx < nrc)
                    def _():
                        fn(kidx * RB)

        if use_init:
            def _init_blk(off):
                pltpu.sync_copy(init_h.at[cid, pl.ds(off, RB)],
                                rows[0].at[pl.ds(0, RB)])
                pltpu.sync_copy(rows[0].at[pl.ds(0, RB)],
                                aggr.at[pl.ds(off, RB)])
            _row_blocks(_init_blk)
        else:
            zv = jnp.zeros((16,), jnp.float32)
            for i in range(RB):
                for j in range(nsl):
                    rows[0][i, pl.ds(j * 16, 16)] = zv

            def _zero_blk(off):
                pltpu.sync_copy(rows[0].at[pl.ds(0, RB)],
                                aggr.at[pl.ds(off, RB)])
            _row_blocks(_zero_blk)
        plsc.subcore_barrier()

        ebase = (0 if col_split else cid * (E // 2)) + sid * (nch * CH)

        def idx_issue(m, b):
            off = ebase + m * CH
            pltpu.async_copy(src_h.at[pl.ds(off, CH)], srcb[b], sem_i[b])
            pltpu.async_copy(dst_h.at[pl.ds(off, CH)], dstb[b], sem_i[b])
            pltpu.async_copy(ea_h.at[pl.ds(off, CH)], eab[b], sem_i[b])

        def idx_wait(b):
            off = ebase
            pltpu.make_async_copy(src_h.at[pl.ds(off, CH)], srcb[b], sem_i[b]).wait()
            pltpu.make_async_copy(dst_h.at[pl.ds(off, CH)], dstb[b], sem_i[b]).wait()
            pltpu.make_async_copy(ea_h.at[pl.ds(off, CH)], eab[b], sem_i[b]).wait()

        def gather_issue(b):
            if col_split:
                @pl.when(cid == 0)
                def _():
                    pltpu.async_copy(t_hs[0].at[srcb[b]], rows[b], sem_g[b])

                @pl.when(cid == 1)
                def _():
                    pltpu.async_copy(t_hs[1].at[srcb[b]], rows[b], sem_g[b])
            else:
                pltpu.async_copy(t_hs[0].at[srcb[b]], rows[b], sem_g[b])

        def gather_wait(b):
            pltpu.make_async_copy(t_hs[0].at[srcb[b]], rows[b], sem_g[b]).wait()

        def scatter_issue(b):
            for g in range(CH // 16):
                sl = pl.ds(g * 16, 16)
                dsts[b][sl] = dstb[b][sl]
            pltpu.async_copy(rows[b], aggr.at[dsts[b]], sem_s[b], add=True)

        def scatter_wait(b):
            pltpu.make_async_copy(rows[b], aggr.at[dsts[b]], sem_s[b]).wait()

        def scale(b):
            for g in range(CH // 16):
                ev = eab[b][pl.ds(g * 16, 16)]
                for l in range(16):
                    i = g * 16 + l
                    e = ev[l]
                    for j in range(nsl):
                        sl = pl.ds(j * 16, 16)
                        rows[b][i, sl] = rows[b][i, sl] * e

        def do_chunk(m, b):
            stat = isinstance(m, int)
            b1 = (b + 1) % NBUF
            b2 = (b + 2) % NBUF
            if not stat or m + 2 < nch:
                idx_issue(m + 2, b2)
            if not stat or m + 1 < nch:
                if not stat or m >= NBUF - 1:
                    scatter_wait(b1)
                idx_wait(b1)
                gather_issue(b1)
            gather_wait(b)
            scale(b)
            scatter_issue(b)

        idx_issue(0, 0)
        idx_issue(1, 1)
        idx_wait(0)
        gather_issue(0)
        front = NBUF - 1
        for m in range(front):
            do_chunk(m, m % NBUF)
        back = ((nch - front - 1) % NBUF) + 1
        nmid = nch - front - back
        assert nmid % NBUF == 0 and nmid >= 0

        def loop_body(it, carry):
            m0 = front + it * NBUF
            for b0 in range(NBUF):
                do_chunk(m0 + b0, (front + b0) % NBUF)
            return carry

        lax.fori_loop(0, nmid // NBUF, loop_body, 0)
        for mt in range(nch - back, nch):
            do_chunk(mt, mt % NBUF)
        for m in range(nch - NBUF, nch):
            scatter_wait(m % NBUF)

        plsc.subcore_barrier()

        def _out_blk(off):
            pltpu.sync_copy(aggr.at[pl.ds(off, RB)], rows[0].at[pl.ds(0, RB)])
            pltpu.sync_copy(rows[0].at[pl.ds(0, RB)],
                            out_h.at[cid, pl.ds(off, RB)])
        _row_blocks(_out_blk)

    k = pl.kernel(
        body,
        out_type=jax.ShapeDtypeStruct((NCORE, N, Dh), jnp.float32),
        mesh=mesh,
        scratch_types=scratch,
    )
    args = tuple(tables) + (src, dst, ea) + ((init,) if use_init else ())
    return k(*args)


_BN = 1000


def _mm(terms, bias, relu):
    nt = len(terms)
    Do = terms[0][1].shape[1]

    def bodyfn(*refs):
        a = refs[:nt]
        w = refs[nt:2 * nt]
        b = refs[2 * nt]
        o = refs[2 * nt + 1]
        acc = jnp.dot(a[0][...], w[0][...], preferred_element_type=jnp.float32)
        for t in range(1, nt):
            acc = acc + jnp.dot(a[t][...], w[t][...],
                                preferred_element_type=jnp.float32)
        acc = acc + b[...]
        if relu:
            acc = jnp.maximum(acc, 0.0)
        o[...] = acc

    in_specs = (
        [pl.BlockSpec((_BN, A.shape[1]), lambda i: (i, 0)) for A, _ in terms]
        + [pl.BlockSpec(W.shape, lambda i: (0, 0)) for _, W in terms]
        + [pl.BlockSpec((1, Do), lambda i: (0, 0))]
    )
    return pl.pallas_call(
        bodyfn,
        grid=(N // _BN,),
        in_specs=in_specs,
        out_specs=pl.BlockSpec((_BN, Do), lambda i: (i, 0)),
        out_shape=jax.ShapeDtypeStruct((N, Do), jnp.float32),
    )(*[A for A, _ in terms], *[W for _, W in terms], bias)


def _bn_stats(pre, gamma2, beta2):
    Dh = pre.shape[1]
    G = N // _BN

    def bodyfn(p, g, bt, o, accs, accq):
        i = pl.program_id(0)

        @pl.when(i == 0)
        def _():
            accs[...] = jnp.zeros_like(accs)
            accq[...] = jnp.zeros_like(accq)

        blk = p[...]
        accs[...] += jnp.sum(blk, axis=0, keepdims=True)
        accq[...] += jnp.sum(blk * blk, axis=0, keepdims=True)

        @pl.when(i == G - 1)
        def _():
            mean = accs[...] * (1.0 / N)
            var = accq[...] * (1.0 / N) - mean * mean
            s = g[...] * lax.rsqrt(var + 1e-5)
            o[...] = jnp.concatenate([s, bt[...] - mean * s], axis=0)

    return pl.pallas_call(
        bodyfn,
        grid=(G,),
        in_specs=[pl.BlockSpec((_BN, Dh), lambda i: (i, 0)),
                  pl.BlockSpec((1, Dh), lambda i: (0, 0)),
                  pl.BlockSpec((1, Dh), lambda i: (0, 0))],
        out_specs=pl.BlockSpec((2, Dh), lambda i: (0, 0)),
        out_shape=jax.ShapeDtypeStruct((2, Dh), jnp.float32),
        scratch_shapes=[pltpu.VMEM((1, Dh), jnp.float32),
                        pltpu.VMEM((1, Dh), jnp.float32)],
    )(pre, gamma2, beta2)


def _bn_apply_mm(p0, p1, st0, st1, w4rel, w4root, b4row):
    def bodyfn(p0r, p1r, s0r, s1r, wr, wo, br, g_o, r_o):
        h0 = jnp.maximum(p0r[...] * s0r[0:1, :] + s0r[1:2, :], 0.0)
        h1 = jnp.maximum(p1r[...] * s1r[0:1, :] + s1r[1:2, :], 0.0)
        g_o[...] = (jnp.dot(h0, wr[0:128, :], preferred_element_type=jnp.float32)
                    + jnp.dot(h1, wr[128:256, :], preferred_element_type=jnp.float32))
        r_o[...] = (jnp.dot(h0, wo[0:128, :], preferred_element_type=jnp.float32)
                    + jnp.dot(h1, wo[128:256, :], preferred_element_type=jnp.float32)
                    + br[...])

    return pl.pallas_call(
        bodyfn,
        grid=(N // _BN,),
        in_specs=[pl.BlockSpec((_BN, 128), lambda i: (i, 0)),
                  pl.BlockSpec((_BN, 128), lambda i: (i, 0)),
                  pl.BlockSpec((2, 128), lambda i: (0, 0)),
                  pl.BlockSpec((2, 128), lambda i: (0, 0)),
                  pl.BlockSpec((256, 128), lambda i: (0, 0)),
                  pl.BlockSpec((256, 128), lambda i: (0, 0)),
                  pl.BlockSpec((1, 128), lambda i: (0, 0))],
        out_specs=[pl.BlockSpec((_BN, 128), lambda i: (i, 0)),
                   pl.BlockSpec((_BN, 128), lambda i: (i, 0))],
        out_shape=[jax.ShapeDtypeStruct((N, 128), jnp.float32),
                   jax.ShapeDtypeStruct((N, 128), jnp.float32)],
    )(p0, p1, st0, st1, w4rel, w4root, b4row)


def _add2(a, b):
    def bodyfn(ar, br, o):
        o[...] = ar[...] + br[...]

    Do = a.shape[1]
    return pl.pallas_call(
        bodyfn,
        grid=(N // _BN,),
        in_specs=[pl.BlockSpec((_BN, Do), lambda i: (i, 0)),
                  pl.BlockSpec((_BN, Do), lambda i: (i, 0))],
        out_specs=pl.BlockSpec((_BN, Do), lambda i: (i, 0)),
        out_shape=jax.ShapeDtypeStruct((N, Do), jnp.float32),
    )(a, b)


def kernel(x, edge_index, edge_attr, W1rel, b1rel, W1root, W2rel, b2rel,
           W2root, W3rel, b3rel, W3root, W4rel, b4rel, W4root, gamma, beta):
    src = edge_index[0]
    dst = edge_index[1]
    f32 = jnp.float32

    a1 = _segsum((x,), src, dst, edge_attr)
    h1 = _mm([(a1[0], W1rel), (a1[1], W1rel), (x, W1root)],
             b1rel.reshape(1, -1), relu=True)

    a2 = _segsum((h1[:, :128], h1[:, 128:]), src, dst, edge_attr)
    h2 = _mm([(a2[0], W2rel[:128]), (a2[1], W2rel[128:]), (h1, W2root)],
             b2rel.reshape(1, -1), relu=True)

    g3 = _mm([(h2, W3rel)], jnp.zeros((1, 256), f32), relu=False)
    root3 = _mm([(h2, W3root)], b3rel.reshape(1, -1), relu=False)
    init3 = jnp.stack([root3[:, :128], root3[:, 128:]])
    p3 = _segsum((g3[:, :128], g3[:, 128:]), src, dst, edge_attr, init=init3)

    st0 = _bn_stats(p3[0], gamma[:128].reshape(1, -1), beta[:128].reshape(1, -1))
    st1 = _bn_stats(p3[1], gamma[128:].reshape(1, -1), beta[128:].reshape(1, -1))
    g4, root4 = _bn_apply_mm(p3[0], p3[1], st0, st1, W4rel, W4root,
                             b4rel.reshape(1, -1))

    zeros128 = jnp.zeros((N, 128), f32)
    init4 = jnp.stack([root4, zeros128])
    o4 = _segsum((g4,), src, dst, edge_attr, init=init4)
    return _add2(o4[0], o4[1])


# speedup vs baseline: 1.0393x; 1.0393x over previous
import functools

import jax
import jax.numpy as jnp
from jax import lax
from jax.experimental import pallas as pl
from jax.experimental.pallas import tpu as pltpu
from jax.experimental.pallas import tpu_sc as plsc

N = 10000
E = 320000
NCORE = 2
NSUB = 16
CH = 80
RB = 80
NBUF = 3


def _pack_edges(src, dst, ea):
    pk = jnp.concatenate([src.reshape(-1, CH), dst.reshape(-1, CH)], axis=1)
    return pk.reshape(-1), ea


def _segsum(tables, pk, eaw, init=None):
    col_split = len(tables) == 2
    Dh = tables[0].shape[1]
    nsl = Dh // 16
    use_init = init is not None
    ec = E if col_split else E // 2
    nch = ec // CH // NSUB
    assert nch * CH * NSUB == ec
    mesh = plsc.VectorSubcoreMesh(core_axis_name="c", subcore_axis_name="s")
    scratch = [
        [pltpu.VMEM((CH, Dh), jnp.float32) for _ in range(NBUF)],
        [pltpu.VMEM((2 * CH,), jnp.int32) for _ in range(NBUF)],
        [pltpu.VMEM((CH,), jnp.float32) for _ in range(NBUF)],
        [pltpu.VMEM((CH,), jnp.int32) for _ in range(NBUF)],
        pltpu.VMEM_SHARED((N, Dh), jnp.float32),
        [pltpu.SemaphoreType.DMA for _ in range(NBUF)],
        [pltpu.SemaphoreType.DMA for _ in range(NBUF)],
        [pltpu.SemaphoreType.DMA for _ in range(NBUF)],
    ]

    def body(*refs):
        t_hs = refs[:len(tables)]
        refs = refs[len(tables):]
        if use_init:
            (pk_h, ea_h, init_h, out_h,
             rows, idxb, eab, dsts, aggr, sem_i, sem_g, sem_s) = refs
        else:
            (pk_h, ea_h, out_h,
             rows, idxb, eab, dsts, aggr, sem_i, sem_g, sem_s) = refs
        cid = lax.axis_index("c")
        sid = lax.axis_index("s")
        nrc = N // RB

        def _row_blocks(fn):
            for m in range((nrc + NSUB - 1) // NSUB):
                kidx = sid + NSUB * m
                if (m + 1) * NSUB <= nrc:
                    fn(kidx * RB)
                else:
                    @pl.when(kidx < nrc)
                    def _():
                        fn(kidx * RB)

        if use_init:
            def _init_blk(off):
                pltpu.sync_copy(init_h.at[cid, pl.ds(off, RB)],
                                rows[0].at[pl.ds(0, RB)])
                pltpu.sync_copy(rows[0].at[pl.ds(0, RB)],
                                aggr.at[pl.ds(off, RB)])
            _row_blocks(_init_blk)
        else:
            zv = jnp.zeros((16,), jnp.float32)
            for i in range(RB):
                for j in range(nsl):
                    rows[0][i, pl.ds(j * 16, 16)] = zv

            def _zero_blk(off):
                pltpu.sync_copy(rows[0].at[pl.ds(0, RB)],
                                aggr.at[pl.ds(off, RB)])
            _row_blocks(_zero_blk)
        plsc.subcore_barrier()

        gbase = (0 if col_split else cid * (ec // CH)) + sid * nch

        def idx_issue(m, b):
            off = (gbase + m) * (2 * CH)
            pltpu.async_copy(pk_h.at[pl.ds(off, 2 * CH)], idxb[b], sem_i[b])
            pltpu.async_copy(ea_h.at[pl.ds((gbase + m) * CH, CH)], eab[b],
                             sem_i[b])

        def idx_wait(b):
            pltpu.make_async_copy(pk_h.at[pl.ds(0, 2 * CH)], idxb[b],
                                  sem_i[b]).wait()
            pltpu.make_async_copy(ea_h.at[pl.ds(0, CH)], eab[b],
                                  sem_i[b]).wait()

        def gather_issue(b):
            idx = idxb[b].at[pl.ds(0, CH)]
            if col_split:
                @pl.when(cid == 0)
                def _():
                    pltpu.async_copy(t_hs[0].at[idx], rows[b], sem_g[b])

                @pl.when(cid == 1)
                def _():
                    pltpu.async_copy(t_hs[1].at[idx], rows[b], sem_g[b])
            else:
                pltpu.async_copy(t_hs[0].at[idx], rows[b], sem_g[b])

        def gather_wait(b):
            idx = idxb[b].at[pl.ds(0, CH)]
            pltpu.make_async_copy(t_hs[0].at[idx], rows[b], sem_g[b]).wait()

        def scatter_issue(b):
            for g in range(CH // 16):
                dsts[b][pl.ds(g * 16, 16)] = idxb[b][pl.ds(CH + g * 16, 16)]
            pltpu.async_copy(rows[b], aggr.at[dsts[b]], sem_s[b], add=True)

        def scatter_wait(b):
            pltpu.make_async_copy(rows[b], aggr.at[dsts[b]], sem_s[b]).wait()

        def scale(b):
            for g in range(CH // 16):
                ev = eab[b][pl.ds(g * 16, 16)]
                for l in range(16):
                    i = g * 16 + l
                    e = ev[l]
                    for j in range(nsl):
                        sl = pl.ds(j * 16, 16)
                        rows[b][i, sl] = rows[b][i, sl] * e

        def do_chunk(m, b):
            stat = isinstance(m, int)
            b1 = (b + 1) % NBUF
            b2 = (b + 2) % NBUF
            if not stat or m + 2 < nch:
                idx_issue(m + 2, b2)
            if not stat or m + 1 < nch:
                if not stat or m >= NBUF - 1:
                    scatter_wait(b1)
                idx_wait(b1)
                gather_issue(b1)
            gather_wait(b)
            scale(b)
            scatter_issue(b)

        idx_issue(0, 0)
        idx_issue(1, 1)
        idx_wait(0)
        gather_issue(0)
        front = NBUF - 1
        for m in range(front):
            do_chunk(m, m % NBUF)
        back = ((nch - front - 1) % NBUF) + 1
        nmid = nch - front - back
        assert nmid % NBUF == 0 and nmid >= 0

        def loop_body(it, carry):
            m0 = front + it * NBUF
            for b0 in range(NBUF):
                do_chunk(m0 + b0, (front + b0) % NBUF)
            return carry

        lax.fori_loop(0, nmid // NBUF, loop_body, 0)
        for mt in range(nch - back, nch):
            do_chunk(mt, mt % NBUF)
        for m in range(nch - NBUF, nch):
            scatter_wait(m % NBUF)

        plsc.subcore_barrier()

        def _out_blk(off):
            pltpu.sync_copy(aggr.at[pl.ds(off, RB)], rows[0].at[pl.ds(0, RB)])
            pltpu.sync_copy(rows[0].at[pl.ds(0, RB)],
                            out_h.at[cid, pl.ds(off, RB)])
        _row_blocks(_out_blk)

    k = pl.kernel(
        body,
        out_type=jax.ShapeDtypeStruct((NCORE, N, Dh), jnp.float32),
        mesh=mesh,
        scratch_types=scratch,
    )
    args = tuple(tables) + (pk, eaw) + ((init,) if use_init else ())
    return k(*args)


_BN = 1000


def _mm(terms, bias, relu):
    nt = len(terms)
    Do = terms[0][1].shape[1]

    def bodyfn(*refs):
        a = refs[:nt]
        w = refs[nt:2 * nt]
        b = refs[2 * nt]
        o = refs[2 * nt + 1]
        acc = jnp.dot(a[0][...], w[0][...], preferred_element_type=jnp.float32)
        for t in range(1, nt):
            acc = acc + jnp.dot(a[t][...], w[t][...],
                                preferred_element_type=jnp.float32)
        acc = acc + b[...]
        if relu:
            acc = jnp.maximum(acc, 0.0)
        o[...] = acc

    in_specs = (
        [pl.BlockSpec((_BN, A.shape[1]), lambda i: (i, 0)) for A, _ in terms]
        + [pl.BlockSpec(W.shape, lambda i: (0, 0)) for _, W in terms]
        + [pl.BlockSpec((1, Do), lambda i: (0, 0))]
    )
    return pl.pallas_call(
        bodyfn,
        grid=(N // _BN,),
        in_specs=in_specs,
        out_specs=pl.BlockSpec((_BN, Do), lambda i: (i, 0)),
        out_shape=jax.ShapeDtypeStruct((N, Do), jnp.float32),
    )(*[A for A, _ in terms], *[W for _, W in terms], bias)


def _bn_stats(pre, gamma2, beta2):
    Dh = pre.shape[1]
    G = N // _BN

    def bodyfn(p, g, bt, o, accs, accq):
        i = pl.program_id(0)

        @pl.when(i == 0)
        def _():
            accs[...] = jnp.zeros_like(accs)
            accq[...] = jnp.zeros_like(accq)

        blk = p[...]
        accs[...] += jnp.sum(blk, axis=0, keepdims=True)
        accq[...] += jnp.sum(blk * blk, axis=0, keepdims=True)

        @pl.when(i == G - 1)
        def _():
            mean = accs[...] * (1.0 / N)
            var = accq[...] * (1.0 / N) - mean * mean
            s = g[...] * lax.rsqrt(var + 1e-5)
            o[...] = jnp.concatenate([s, bt[...] - mean * s], axis=0)

    return pl.pallas_call(
        bodyfn,
        grid=(G,),
        in_specs=[pl.BlockSpec((_BN, Dh), lambda i: (i, 0)),
                  pl.BlockSpec((1, Dh), lambda i: (0, 0)),
                  pl.BlockSpec((1, Dh), lambda i: (0, 0))],
        out_specs=pl.BlockSpec((2, Dh), lambda i: (0, 0)),
        out_shape=jax.ShapeDtypeStruct((2, Dh), jnp.float32),
        scratch_shapes=[pltpu.VMEM((1, Dh), jnp.float32),
                        pltpu.VMEM((1, Dh), jnp.float32)],
    )(pre, gamma2, beta2)


def _bn_apply_mm(p0, p1, st0, st1, w4rel, w4root, b4row):
    def bodyfn(p0r, p1r, s0r, s1r, wr, wo, br, g_o, r_o):
        h0 = jnp.maximum(p0r[...] * s0r[0:1, :] + s0r[1:2, :], 0.0)
        h1 = jnp.maximum(p1r[...] * s1r[0:1, :] + s1r[1:2, :], 0.0)
        g_o[...] = (jnp.dot(h0, wr[0:128, :], preferred_element_type=jnp.float32)
                    + jnp.dot(h1, wr[128:256, :], preferred_element_type=jnp.float32))
        r_o[...] = (jnp.dot(h0, wo[0:128, :], preferred_element_type=jnp.float32)
                    + jnp.dot(h1, wo[128:256, :], preferred_element_type=jnp.float32)
                    + br[...])

    return pl.pallas_call(
        bodyfn,
        grid=(N // _BN,),
        in_specs=[pl.BlockSpec((_BN, 128), lambda i: (i, 0)),
                  pl.BlockSpec((_BN, 128), lambda i: (i, 0)),
                  pl.BlockSpec((2, 128), lambda i: (0, 0)),
                  pl.BlockSpec((2, 128), lambda i: (0, 0)),
                  pl.BlockSpec((256, 128), lambda i: (0, 0)),
                  pl.BlockSpec((256, 128), lambda i: (0, 0)),
                  pl.BlockSpec((1, 128), lambda i: (0, 0))],
        out_specs=[pl.BlockSpec((_BN, 128), lambda i: (i, 0)),
                   pl.BlockSpec((_BN, 128), lambda i: (i, 0))],
        out_shape=[jax.ShapeDtypeStruct((N, 128), jnp.float32),
                   jax.ShapeDtypeStruct((N, 128), jnp.float32)],
    )(p0, p1, st0, st1, w4rel, w4root, b4row)


def _add2(a, b):
    def bodyfn(ar, br, o):
        o[...] = ar[...] + br[...]

    Do = a.shape[1]
    return pl.pallas_call(
        bodyfn,
        grid=(N // _BN,),
        in_specs=[pl.BlockSpec((_BN, Do), lambda i: (i, 0)),
                  pl.BlockSpec((_BN, Do), lambda i: (i, 0))],
        out_specs=pl.BlockSpec((_BN, Do), lambda i: (i, 0)),
        out_shape=jax.ShapeDtypeStruct((N, Do), jnp.float32),
    )(a, b)


def kernel(x, edge_index, edge_attr, W1rel, b1rel, W1root, W2rel, b2rel,
           W2root, W3rel, b3rel, W3root, W4rel, b4rel, W4root, gamma, beta):
    src = edge_index[0]
    dst = edge_index[1]
    f32 = jnp.float32
    pk, eaw = _pack_edges(src, dst, edge_attr)

    a1 = _segsum((x,), pk, eaw)
    h1 = _mm([(a1[0], W1rel), (a1[1], W1rel), (x, W1root)],
             b1rel.reshape(1, -1), relu=True)

    a2 = _segsum((h1[:, :128], h1[:, 128:]), pk, eaw)
    h2 = _mm([(a2[0], W2rel[:128]), (a2[1], W2rel[128:]), (h1, W2root)],
             b2rel.reshape(1, -1), relu=True)

    g3 = _mm([(h2, W3rel)], jnp.zeros((1, 256), f32), relu=False)
    root3 = _mm([(h2, W3root)], b3rel.reshape(1, -1), relu=False)
    init3 = jnp.stack([root3[:, :128], root3[:, 128:]])
    p3 = _segsum((g3[:, :128], g3[:, 128:]), pk, eaw, init=init3)

    st0 = _bn_stats(p3[0], gamma[:128].reshape(1, -1), beta[:128].reshape(1, -1))
    st1 = _bn_stats(p3[1], gamma[128:].reshape(1, -1), beta[128:].reshape(1, -1))
    g4, root4 = _bn_apply_mm(p3[0], p3[1], st0, st1, W4rel, W4root,
                             b4rel.reshape(1, -1))

    zeros128 = jnp.zeros((N, 128), f32)
    init4 = jnp.stack([root4, zeros128])
    o4 = _segsum((g4,), pk, eaw, init=init4)
    return _add2(o4[0], o4[1])


# speedup vs baseline: 1.0411x; 1.0018x over previous
import functools

import jax
import jax.numpy as jnp
from jax import lax
from jax.experimental import pallas as pl
from jax.experimental.pallas import tpu as pltpu
from jax.experimental.pallas import tpu_sc as plsc

N = 10000
E = 320000
NCORE = 2
NSUB = 16
CH = 80
RB = 80
NBUF = 3


def _pack_edges(src, dst, ea):
    pk = jnp.concatenate([src.reshape(-1, CH), dst.reshape(-1, CH)], axis=1)
    return pk.reshape(-1), ea


def _segsum(tables, pk, eaw, init=None):
    col_split = len(tables) == 2
    Dh = tables[0].shape[1]
    nsl = Dh // 16
    use_init = init is not None
    ec = E if col_split else E // 2
    nch = ec // CH // NSUB
    assert nch * CH * NSUB == ec
    mesh = plsc.VectorSubcoreMesh(core_axis_name="c", subcore_axis_name="s")
    scratch = [
        [pltpu.VMEM((CH, Dh), jnp.float32) for _ in range(NBUF)],
        [pltpu.VMEM((2 * CH,), jnp.int32) for _ in range(NBUF)],
        [pltpu.VMEM((CH,), jnp.float32) for _ in range(NBUF)],
        [pltpu.VMEM((CH,), jnp.int32) for _ in range(NBUF)],
        pltpu.VMEM_SHARED((N, Dh), jnp.float32),
        [pltpu.SemaphoreType.DMA for _ in range(NBUF)],
        [pltpu.SemaphoreType.DMA for _ in range(NBUF)],
        [pltpu.SemaphoreType.DMA for _ in range(NBUF)],
    ]

    def body(*refs):
        t_hs = refs[:len(tables)]
        refs = refs[len(tables):]
        if use_init:
            (pk_h, ea_h, init_h, out_h,
             rows, idxb, eab, dsts, aggr, sem_i, sem_g, sem_s) = refs
        else:
            (pk_h, ea_h, out_h,
             rows, idxb, eab, dsts, aggr, sem_i, sem_g, sem_s) = refs
        cid = lax.axis_index("c")
        sid = lax.axis_index("s")
        nrc = N // RB

        def _row_blocks(fn):
            for m in range((nrc + NSUB - 1) // NSUB):
                kidx = sid + NSUB * m
                if (m + 1) * NSUB <= nrc:
                    fn(kidx * RB)
                else:
                    @pl.when(kidx < nrc)
                    def _():
                        fn(kidx * RB)

        if use_init:
            def _init_blk(off):
                pltpu.sync_copy(init_h.at[cid, pl.ds(off, RB)],
                                rows[0].at[pl.ds(0, RB)])
                pltpu.sync_copy(rows[0].at[pl.ds(0, RB)],
                                aggr.at[pl.ds(off, RB)])
            _row_blocks(_init_blk)
        else:
            zv = jnp.zeros((16,), jnp.float32)
            for i in range(RB):
                for j in range(nsl):
                    rows[0][i, pl.ds(j * 16, 16)] = zv

            def _zero_blk(off):
                pltpu.sync_copy(rows[0].at[pl.ds(0, RB)],
                                aggr.at[pl.ds(off, RB)])
            _row_blocks(_zero_blk)
        plsc.subcore_barrier()

        gbase = (0 if col_split else cid * (ec // CH)) + sid * nch

        def idx_issue(m, b):
            off = (gbase + m) * (2 * CH)
            pltpu.async_copy(pk_h.at[pl.ds(off, 2 * CH)], idxb[b], sem_i[b])
            pltpu.async_copy(ea_h.at[pl.ds((gbase + m) * CH, CH)], eab[b],
                             sem_i[b])

        def idx_wait(b):
            pltpu.make_async_copy(pk_h.at[pl.ds(0, 2 * CH)], idxb[b],
                                  sem_i[b]).wait()
            pltpu.make_async_copy(ea_h.at[pl.ds(0, CH)], eab[b],
                                  sem_i[b]).wait()

        def gather_issue(b):
            idx = idxb[b].at[pl.ds(0, CH)]
            if col_split:
                @pl.when(cid == 0)
                def _():
                    pltpu.async_copy(t_hs[0].at[idx], rows[b], sem_g[b])

                @pl.when(cid == 1)
                def _():
                    pltpu.async_copy(t_hs[1].at[idx], rows[b], sem_g[b])
            else:
                pltpu.async_copy(t_hs[0].at[idx], rows[b], sem_g[b])

        def gather_wait(b):
            idx = idxb[b].at[pl.ds(0, CH)]
            pltpu.make_async_copy(t_hs[0].at[idx], rows[b], sem_g[b]).wait()

        def scatter_issue(b):
            for g in range(CH // 16):
                dsts[b][pl.ds(g * 16, 16)] = idxb[b][pl.ds(CH + g * 16, 16)]
            pltpu.async_copy(rows[b], aggr.at[dsts[b]], sem_s[b], add=True)

        def scatter_wait(b):
            pltpu.make_async_copy(rows[b], aggr.at[dsts[b]], sem_s[b]).wait()

        def scale(b):
            for g in range(CH // 16):
                ev = eab[b][pl.ds(g * 16, 16)]
                for l in range(16):
                    i = g * 16 + l
                    e = ev[l]
                    for j in range(nsl):
                        sl = pl.ds(j * 16, 16)
                        rows[b][i, sl] = rows[b][i, sl] * e

        def do_chunk(m, b):
            stat = isinstance(m, int)
            b1 = (b + 1) % NBUF
            b2 = (b + 2) % NBUF
            if not stat or m + 2 < nch:
                idx_issue(m + 2, b2)
            if not stat or m + 1 < nch:
                if not stat or m >= NBUF - 1:
                    scatter_wait(b1)
                idx_wait(b1)
                gather_issue(b1)
            gather_wait(b)
            scale(b)
            scatter_issue(b)

        idx_issue(0, 0)
        idx_issue(1, 1)
        idx_wait(0)
        gather_issue(0)
        front = NBUF - 1
        for m in range(front):
            do_chunk(m, m % NBUF)
        back = ((nch - front - 1) % NBUF) + 1
        nmid = nch - front - back
        assert nmid % NBUF == 0 and nmid >= 0

        def loop_body(it, carry):
            m0 = front + it * NBUF
            for b0 in range(NBUF):
                do_chunk(m0 + b0, (front + b0) % NBUF)
            return carry

        lax.fori_loop(0, nmid // NBUF, loop_body, 0)
        for mt in range(nch - back, nch):
            do_chunk(mt, mt % NBUF)
        for m in range(nch - NBUF, nch):
            scatter_wait(m % NBUF)

        plsc.subcore_barrier()

        def _out_blk(off):
            pltpu.sync_copy(aggr.at[pl.ds(off, RB)], rows[0].at[pl.ds(0, RB)])
            pltpu.sync_copy(rows[0].at[pl.ds(0, RB)],
                            out_h.at[cid, pl.ds(off, RB)])
        _row_blocks(_out_blk)

    k = pl.kernel(
        body,
        out_type=jax.ShapeDtypeStruct((NCORE, N, Dh), jnp.float32),
        mesh=mesh,
        scratch_types=scratch,
    )
    args = tuple(tables) + (pk, eaw) + ((init,) if use_init else ())
    return k(*args)


_BN = 1000


def _mm(terms, bias, relu):
    nt = len(terms)
    Do = terms[0][1].shape[1]

    def bodyfn(*refs):
        a = refs[:nt]
        w = refs[nt:2 * nt]
        b = refs[2 * nt]
        o = refs[2 * nt + 1]
        acc = jnp.dot(a[0][...], w[0][...], preferred_element_type=jnp.float32)
        for t in range(1, nt):
            acc = acc + jnp.dot(a[t][...], w[t][...],
                                preferred_element_type=jnp.float32)
        acc = acc + b[...]
        if relu:
            acc = jnp.maximum(acc, 0.0)
        o[...] = acc

    in_specs = (
        [pl.BlockSpec((_BN, A.shape[1]), lambda i: (i, 0)) for A, _ in terms]
        + [pl.BlockSpec(W.shape, lambda i: (0, 0)) for _, W in terms]
        + [pl.BlockSpec((1, Do), lambda i: (0, 0))]
    )
    return pl.pallas_call(
        bodyfn,
        grid=(N // _BN,),
        in_specs=in_specs,
        out_specs=pl.BlockSpec((_BN, Do), lambda i: (i, 0)),
        out_shape=jax.ShapeDtypeStruct((N, Do), jnp.float32),
    )(*[A for A, _ in terms], *[W for _, W in terms], bias)


def _mm2(a, w1, w2, bias2):
    K = a.shape[1]
    Do = w1.shape[1]

    def bodyfn(ar, w1r, w2r, br, o1, o2):
        blk = ar[...]
        o1[...] = jnp.dot(blk, w1r[...], preferred_element_type=jnp.float32)
        o2[...] = jnp.dot(blk, w2r[...],
                          preferred_element_type=jnp.float32) + br[...]

    return pl.pallas_call(
        bodyfn,
        grid=(N // _BN,),
        in_specs=[pl.BlockSpec((_BN, K), lambda i: (i, 0)),
                  pl.BlockSpec((K, Do), lambda i: (0, 0)),
                  pl.BlockSpec((K, Do), lambda i: (0, 0)),
                  pl.BlockSpec((1, Do), lambda i: (0, 0))],
        out_specs=[pl.BlockSpec((_BN, Do), lambda i: (i, 0)),
                   pl.BlockSpec((_BN, Do), lambda i: (i, 0))],
        out_shape=[jax.ShapeDtypeStruct((N, Do), jnp.float32),
                   jax.ShapeDtypeStruct((N, Do), jnp.float32)],
    )(a, w1, w2, bias2)


def _bn_stats(pre, gamma2, beta2):
    Dh = pre.shape[1]
    G = N // _BN

    def bodyfn(p, g, bt, o, accs, accq):
        i = pl.program_id(0)

        @pl.when(i == 0)
        def _():
            accs[...] = jnp.zeros_like(accs)
            accq[...] = jnp.zeros_like(accq)

        blk = p[...]

        @pl.when(i < G)
        def _():
            accs[...] += jnp.sum(blk, axis=0, keepdims=True)

        @pl.when(i == G - 1)
        def _():
            accs[...] *= (1.0 / N)

        @pl.when(i >= G)
        def _():
            d = blk - accs[...]
            accq[...] += jnp.sum(d * d, axis=0, keepdims=True)

        @pl.when(i == 2 * G - 1)
        def _():
            mean = accs[...]
            var = accq[...] * (1.0 / N)
            s = g[...] * lax.rsqrt(var + 1e-5)
            o[...] = jnp.concatenate([s, bt[...] - mean * s], axis=0)

    return pl.pallas_call(
        bodyfn,
        grid=(2 * G,),
        in_specs=[pl.BlockSpec((_BN, Dh), lambda i: (i % G, 0)),
                  pl.BlockSpec((1, Dh), lambda i: (0, 0)),
                  pl.BlockSpec((1, Dh), lambda i: (0, 0))],
        out_specs=pl.BlockSpec((2, Dh), lambda i: (0, 0)),
        out_shape=jax.ShapeDtypeStruct((2, Dh), jnp.float32),
        scratch_shapes=[pltpu.VMEM((1, Dh), jnp.float32),
                        pltpu.VMEM((1, Dh), jnp.float32)],
    )(pre, gamma2, beta2)


def _bn_apply_mm(p0, p1, st0, st1, w4rel, w4root, b4row):
    def bodyfn(p0r, p1r, s0r, s1r, wr, wo, br, g_o, r_o):
        h0 = jnp.maximum(p0r[...] * s0r[0:1, :] + s0r[1:2, :], 0.0)
        h1 = jnp.maximum(p1r[...] * s1r[0:1, :] + s1r[1:2, :], 0.0)
        g_o[...] = (jnp.dot(h0, wr[0:128, :], preferred_element_type=jnp.float32)
                    + jnp.dot(h1, wr[128:256, :], preferred_element_type=jnp.float32))
        r_o[...] = (jnp.dot(h0, wo[0:128, :], preferred_element_type=jnp.float32)
                    + jnp.dot(h1, wo[128:256, :], preferred_element_type=jnp.float32)
                    + br[...])

    return pl.pallas_call(
        bodyfn,
        grid=(N // _BN,),
        in_specs=[pl.BlockSpec((_BN, 128), lambda i: (i, 0)),
                  pl.BlockSpec((_BN, 128), lambda i: (i, 0)),
                  pl.BlockSpec((2, 128), lambda i: (0, 0)),
                  pl.BlockSpec((2, 128), lambda i: (0, 0)),
                  pl.BlockSpec((256, 128), lambda i: (0, 0)),
                  pl.BlockSpec((256, 128), lambda i: (0, 0)),
                  pl.BlockSpec((1, 128), lambda i: (0, 0))],
        out_specs=[pl.BlockSpec((_BN, 128), lambda i: (i, 0)),
                   pl.BlockSpec((_BN, 128), lambda i: (i, 0))],
        out_shape=[jax.ShapeDtypeStruct((N, 128), jnp.float32),
                   jax.ShapeDtypeStruct((N, 128), jnp.float32)],
    )(p0, p1, st0, st1, w4rel, w4root, b4row)


def _add2(a, b):
    def bodyfn(ar, br, o):
        o[...] = ar[...] + br[...]

    Do = a.shape[1]
    return pl.pallas_call(
        bodyfn,
        grid=(N // _BN,),
        in_specs=[pl.BlockSpec((_BN, Do), lambda i: (i, 0)),
                  pl.BlockSpec((_BN, Do), lambda i: (i, 0))],
        out_specs=pl.BlockSpec((_BN, Do), lambda i: (i, 0)),
        out_shape=jax.ShapeDtypeStruct((N, Do), jnp.float32),
    )(a, b)


def kernel(x, edge_index, edge_attr, W1rel, b1rel, W1root, W2rel, b2rel,
           W2root, W3rel, b3rel, W3root, W4rel, b4rel, W4root, gamma, beta):
    src = edge_index[0]
    dst = edge_index[1]
    f32 = jnp.float32
    pk, eaw = _pack_edges(src, dst, edge_attr)

    a1 = _segsum((x,), pk, eaw)
    h1 = _mm([(a1[0], W1rel), (a1[1], W1rel), (x, W1root)],
             b1rel.reshape(1, -1), relu=True)

    a2 = _segsum((h1[:, :128], h1[:, 128:]), pk, eaw)
    h2 = _mm([(a2[0], W2rel[:128]), (a2[1], W2rel[128:]), (h1, W2root)],
             b2rel.reshape(1, -1), relu=True)

    g3, root3 = _mm2(h2, W3rel, W3root, b3rel.reshape(1, -1))
    init3 = jnp.stack([root3[:, :128], root3[:, 128:]])
    p3 = _segsum((g3[:, :128], g3[:, 128:]), pk, eaw, init=init3)

    st0 = _bn_stats(p3[0], gamma[:128].reshape(1, -1), beta[:128].reshape(1, -1))
    st1 = _bn_stats(p3[1], gamma[128:].reshape(1, -1), beta[128:].reshape(1, -1))
    g4, root4 = _bn_apply_mm(p3[0], p3[1], st0, st1, W4rel, W4root,
                             b4rel.reshape(1, -1))

    zeros128 = jnp.zeros((N, 128), f32)
    init4 = jnp.stack([root4, zeros128])
    o4 = _segsum((g4,), pk, eaw, init=init4)
    return _add2(o4[0], o4[1])
